# baseline (speedup 1.0000x reference)
import functools

import jax
import jax.numpy as jnp
from jax import lax
from jax.experimental import pallas as pl
from jax.experimental.pallas import tpu as pltpu
from jax.experimental.pallas import tpu_sc as plsc

_B = 4096
_DC = 128
_D = 64
_DP = 128
_K = 200
_KP = 208
_NC = 2
_NS = 16
_NW = _NC * _NS
_BPW = _B // _NW
_NBUF = 2
_OFFS = tuple(list(range(0, 192, 16)) + [184])


def _ctx_body(x_ref, w_ref, b_ref, o_ref):
    o_ref[...] = jnp.maximum(
        jnp.dot(x_ref[...], w_ref[...], preferred_element_type=jnp.float32)
        + b_ref[...],
        0.0,
    )


def _context_mlp(x, W, b):
    blk = 512
    return pl.pallas_call(
        _ctx_body,
        grid=(_B // blk,),
        in_specs=[
            pl.BlockSpec((blk, _DC), lambda i: (i, 0)),
            pl.BlockSpec((_DC, _D), lambda i: (0, 0)),
            pl.BlockSpec((1, _D), lambda i: (0, 0)),
        ],
        out_specs=pl.BlockSpec((blk, _D), lambda i: (i, 0)),
        out_shape=jax.ShapeDtypeStruct((_B, _D), jnp.float32),
    )(x, W, b.reshape(1, _D))


def _sc_body(table_hbm, ak_hbm, ctx_hbm, out_hbm,
             idx_v, ctx_v, rows_v, scores_v,
             out_v0, out_v1,
             gsem0, gsem1, osem0, osem1):
    out_vs = (out_v0, out_v1)
    gsems = (gsem0, gsem1)
    osems = (osem0, osem1)
    wid = lax.axis_index("s") * _NC + lax.axis_index("c")
    base = wid * _BPW

    pltpu.sync_copy(ak_hbm.at[pl.ds(base, _BPW)], idx_v)
    pltpu.sync_copy(ctx_hbm.at[pl.ds(base, _BPW)], ctx_v)

    lane = lax.iota(jnp.int32, 16)

    def issue_gather(b, p):
        for off in _OFFS:
            idx16 = lax.shift_right_logical(idx_v[b, pl.ds(off, 16)], 1)
            pltpu.async_copy(
                table_hbm.at[idx16],
                rows_v.at[p, pl.ds(off, 16)],
                gsems[p],
            )

    def wait_gather(b, p):
        for off in _OFFS:
            pltpu.make_async_copy(
                table_hbm.at[lax.shift_right_logical(idx_v[b, pl.ds(off, 16)], 1)],
                rows_v.at[p, pl.ds(off, 16)],
                gsems[p],
            ).wait()

    for p in range(_NBUF):
        issue_gather(p, p)

    @pl.loop(0, _BPW)
    def _outer(b):
        p_dyn = lax.rem(b, _NBUF)
        for p in range(_NBUF):

            @pl.when(p_dyn == p)
            def _():
                wait_gather(b, p)

                t = lax.rem(lax.div(b, 8), 2)
                for q in range(2):

                    @pl.when(t == q)
                    def _():
                        o = out_vs[q]
                        osem = osems[q]
                        r = lax.rem(b, 8)

                        c0 = ctx_v[b, pl.ds(0, 16)]
                        c1 = ctx_v[b, pl.ds(16, 16)]
                        c2 = ctx_v[b, pl.ds(32, 16)]
                        c3 = ctx_v[b, pl.ds(48, 16)]

                        m = jnp.full((16,), -1e30, jnp.float32)
                        for off in _OFFS:
                            iv = idx_v[b, pl.ds(off, 16)]
                            v = jnp.zeros((16,), jnp.float32)
                            for kk in range(16):
                                k = off + kk
                                h = (iv[kk] & 1) * 64
                                acc = rows_v[p, k, pl.ds(h, 16)] * c0
                                acc = acc + rows_v[p, k, pl.ds(h + 16, 16)] * c1
                                acc = acc + rows_v[p, k, pl.ds(h + 32, 16)] * c2
                                acc = acc + rows_v[p, k, pl.ds(h + 48, 16)] * c3
                                v = jnp.where(lane == kk, jnp.sum(acc), v)
                            scores_v[pl.ds(off, 16)] = v
                            m = jnp.maximum(m, v)

                        mx = jnp.max(m)

                        @pl.when(b + _NBUF < _BPW)
                        def _():
                            issue_gather(b + _NBUF, p)

                        @pl.when((r == 0) & (b >= 16))
                        def _():
                            pltpu.make_async_copy(
                                o,
                                out_hbm.at[pl.ds(pl.multiple_of(base + b - 16, 8), 8)],
                                osem,
                            ).wait()

                        tot = jnp.zeros((16,), jnp.float32)
                        for off in _OFFS:
                            e = jnp.exp(scores_v[pl.ds(off, 16)] - mx)
                            if off == 184:
                                tot = tot + jnp.where(lane >= 8, e, 0.0)
                            else:
                                tot = tot + e

                        tvec = jnp.zeros((16,), jnp.float32) + jnp.sum(tot)

                        for off in _OFFS:
                            e = jnp.exp(scores_v[pl.ds(off, 16)] - mx)
                            o[r, pl.ds(off, 16)] = e / tvec

                        @pl.when(r == 7)
                        def _():
                            pltpu.async_copy(
                                o,
                                out_hbm.at[pl.ds(pl.multiple_of(base + b - 7, 8), 8)],
                                osem,
                            )

    for q in range(2):
        pltpu.make_async_copy(
            out_vs[q],
            out_hbm.at[pl.ds(pl.multiple_of(base + _BPW - 16 + 8 * q, 8), 8)],
            osems[q],
        ).wait()


_sc_kernel = functools.partial(
    pl.kernel,
    out_type=jax.ShapeDtypeStruct((_B, _K), jnp.float32),
    mesh=plsc.VectorSubcoreMesh(core_axis_name="c", subcore_axis_name="s"),
    compiler_params=pltpu.CompilerParams(
        needs_layout_passes=False, use_tc_tiling_on_sc=True
    ),
    scratch_types=[
        pltpu.VMEM((_BPW, _K), jnp.int32),
        pltpu.VMEM((_BPW, _D), jnp.float32),
        pltpu.VMEM((_NBUF, _K, _DP), jnp.float32),
        pltpu.VMEM((_KP,), jnp.float32),
        pltpu.VMEM((8, _K), jnp.float32),
        pltpu.VMEM((8, _K), jnp.float32),
    ] + [pltpu.SemaphoreType.DMA] * 4,
)(_sc_body)


def kernel(x, A_k, W, b, table):
    ctx = _context_mlp(x, W, b)
    ak = A_k.astype(jnp.int32)
    tp = table.reshape(_B * 0 + 500000, _DP)
    return _sc_kernel(tp, ak, ctx)


# speedup vs baseline: 1.4342x; 1.4342x over previous
import functools

import jax
import jax.numpy as jnp
from jax import lax
from jax.experimental import pallas as pl
from jax.experimental.pallas import tpu as pltpu
from jax.experimental.pallas import tpu_sc as plsc

_B = 4096
_DC = 128
_D = 64
_K = 200
_KP = 208
_G = _KP // 16
_NC = 2
_NS = 16
_NW = _NC * _NS
_BPW = _B // _NW
_BG = 2
_NG = _BPW // _BG
_GR = _BG * _K
_NBUF = 3


def _ctx_body(x_ref, w_ref, b_ref, o_ref):
    o_ref[...] = jnp.maximum(
        jnp.dot(x_ref[...], w_ref[...], preferred_element_type=jnp.float32)
        + b_ref[...],
        0.0,
    )


def _context_mlp(x, W, b):
    blk = 512
    return pl.pallas_call(
        _ctx_body,
        grid=(_B // blk,),
        in_specs=[
            pl.BlockSpec((blk, _DC), lambda i: (i, 0)),
            pl.BlockSpec((_DC, _D), lambda i: (0, 0)),
            pl.BlockSpec((1, _D), lambda i: (0, 0)),
        ],
        out_specs=pl.BlockSpec((blk, _D), lambda i: (i, 0)),
        out_shape=jax.ShapeDtypeStruct((_B, _D), jnp.float32),
    )(x, W, b.reshape(1, _D))


def _sc_body(table_hbm, ak_hbm, ctx_hbm, out_hbm,
             idx_v, ctx_v, rows_v, scores_v,
             out_v0, out_v1,
             gsem0, gsem1, gsem2, osem0, osem1):
    out_vs = (out_v0, out_v1)
    gsems = (gsem0, gsem1, gsem2)
    osems = (osem0, osem1)
    wid = lax.axis_index("s") * _NC + lax.axis_index("c")

    pltpu.sync_copy(ak_hbm.at[pl.ds(wid * _NG, _NG)], idx_v)
    pltpu.sync_copy(ctx_hbm.at[pl.ds(wid * _BPW, _BPW)], ctx_v)

    lane = lax.iota(jnp.int32, 16)

    def issue_gather(g, p):
        for r in range(_GR // 16):
            idx16 = idx_v[g, pl.ds(r * 16, 16)]
            pltpu.async_copy(
                table_hbm.at[idx16],
                rows_v.at[p, pl.ds(r * 16, 16)],
                gsems[p],
            )

    def wait_gather(g, p):
        pltpu.make_async_copy(
            table_hbm.at[idx_v.at[g]], rows_v.at[p], gsems[p]
        ).wait()

    for p in range(_NBUF):
        issue_gather(p, p)

    @pl.loop(0, _NG)
    def _outer(g):
        p_dyn = lax.rem(g, _NBUF)
        for p in range(_NBUF):

            @pl.when(p_dyn == p)
            def _():
                wait_gather(g, p)

                for t in range(_BG):
                    b = g * _BG + t
                    o = out_vs[t]
                    osem = osems[t]

                    c0 = ctx_v[b, pl.ds(0, 16)]
                    c1 = ctx_v[b, pl.ds(16, 16)]
                    c2 = ctx_v[b, pl.ds(32, 16)]
                    c3 = ctx_v[b, pl.ds(48, 16)]

                    @pl.loop(
                        0, _G,
                        init_carry=jnp.full((16,), -1e30, jnp.float32),
                    )
                    def _groups(g16, m):
                        v = jnp.zeros((16,), jnp.float32)
                        for kk in range(16):
                            k = t * _K + g16 * 16 + kk
                            acc = rows_v[p, k, pl.ds(0, 16)] * c0
                            acc = acc + rows_v[p, k, pl.ds(16, 16)] * c1
                            acc = acc + rows_v[p, k, pl.ds(32, 16)] * c2
                            acc = acc + rows_v[p, k, pl.ds(48, 16)] * c3
                            v = jnp.where(lane == kk, jnp.sum(acc), v)
                        v = jnp.where(g16 * 16 + lane < _K, v, -1e30)
                        scores_v[pl.ds(g16 * 16, 16)] = v
                        return jnp.maximum(m, v)

                    mx = jnp.max(_groups)

                    @pl.when(g > 0)
                    def _():
                        pltpu.make_async_copy(
                            o.at[pl.ds(0, _K)],
                            out_hbm.at[wid * _BPW + b - _BG],
                            osem,
                        ).wait()

                    @pl.loop(
                        0, _G, init_carry=jnp.zeros((16,), jnp.float32)
                    )
                    def _expsum(g16, tot):
                        e = jnp.exp(scores_v[pl.ds(g16 * 16, 16)] - mx)
                        o[pl.ds(g16 * 16, 16)] = e
                        return tot + e

                    tvec = jnp.zeros((16,), jnp.float32) + jnp.sum(_expsum)

                    @pl.loop(0, _G)
                    def _scale(g16):
                        o[pl.ds(g16 * 16, 16)] = o[pl.ds(g16 * 16, 16)] / tvec

                    pltpu.async_copy(
                        o.at[pl.ds(0, _K)],
                        out_hbm.at[wid * _BPW + b],
                        osem,
                    )

                @pl.when(g + _NBUF < _NG)
                def _():
                    issue_gather(g + _NBUF, p)

    for t in range(_BG):
        pltpu.make_async_copy(
            out_vs[t].at[pl.ds(0, _K)],
            out_hbm.at[wid * _BPW + _BPW - _BG + t],
            osems[t],
        ).wait()


_sc_kernel = functools.partial(
    pl.kernel,
    out_type=jax.ShapeDtypeStruct((_B, _K), jnp.float32),
    mesh=plsc.VectorSubcoreMesh(core_axis_name="c", subcore_axis_name="s"),
    compiler_params=pltpu.CompilerParams(
        needs_layout_passes=False, use_tc_tiling_on_sc=False
    ),
    scratch_types=[
        pltpu.VMEM((_NG, _GR), jnp.int32),
        pltpu.VMEM((_BPW, _D), jnp.float32),
        pltpu.VMEM((_NBUF, _GR, _D), jnp.float32),
        pltpu.VMEM((_KP,), jnp.float32),
        pltpu.VMEM((_KP,), jnp.float32),
        pltpu.VMEM((_KP,), jnp.float32),
    ] + [pltpu.SemaphoreType.DMA] * 5,
)(_sc_body)


def kernel(x, A_k, W, b, table):
    ctx = _context_mlp(x, W, b)
    ak = A_k.astype(jnp.int32).reshape(_B // _BG, _GR)
    return _sc_kernel(table, ak, ctx)


# speedup vs baseline: 1.4883x; 1.0378x over previous
import functools

import jax
import jax.numpy as jnp
from jax import lax
from jax.experimental import pallas as pl
from jax.experimental.pallas import tpu as pltpu
from jax.experimental.pallas import tpu_sc as plsc

_B = 4096
_DC = 128
_D = 64
_K = 200
_KP = 208
_G = _KP // 16
_CH = 100
_NC = 2
_NS = 16
_NW = _NC * _NS
_BPW = _B // _NW
_NBUF = 4


def _ctx_body(x_ref, w_ref, b_ref, o_ref):
    o_ref[...] = jnp.maximum(
        jnp.dot(x_ref[...], w_ref[...], preferred_element_type=jnp.float32)
        + b_ref[...],
        0.0,
    )


def _context_mlp(x, W, b):
    blk = 512
    return pl.pallas_call(
        _ctx_body,
        grid=(_B // blk,),
        in_specs=[
            pl.BlockSpec((blk, _DC), lambda i: (i, 0)),
            pl.BlockSpec((_DC, _D), lambda i: (0, 0)),
            pl.BlockSpec((1, _D), lambda i: (0, 0)),
        ],
        out_specs=pl.BlockSpec((blk, _D), lambda i: (i, 0)),
        out_shape=jax.ShapeDtypeStruct((_B, _D), jnp.float32),
    )(x, W, b.reshape(1, _D))


def _sc_body(table_hbm, ak_hbm, ctx_hbm, out_hbm,
             idx_v, ctx_v, rows_v, scores_v,
             out_v0, out_v1, out_v2, out_v3,
             gsem0, gsem1, gsem2, gsem3, osem0, osem1, osem2, osem3):
    out_vs = (out_v0, out_v1, out_v2, out_v3)
    wid = lax.axis_index("s") * _NC + lax.axis_index("c")
    base = wid * _BPW

    pltpu.sync_copy(ak_hbm.at[pl.ds(base, _BPW)], idx_v)
    pltpu.sync_copy(ctx_hbm.at[pl.ds(base, _BPW)], ctx_v)

    gsems = (gsem0, gsem1, gsem2, gsem3)
    osems = (osem0, osem1, osem2, osem3)
    lane = lax.iota(jnp.int32, 16)

    def issue_gather(b, p):
        for j in range(2):
            pltpu.async_copy(
                table_hbm.at[idx_v.at[b, j]],
                rows_v.at[p, pl.ds(j * _CH, _CH)],
                gsems[p],
            )

    def wait_gather(b, p):
        for j in range(2):
            pltpu.make_async_copy(
                table_hbm.at[idx_v.at[b, j]],
                rows_v.at[p, pl.ds(j * _CH, _CH)],
                gsems[p],
            ).wait()

    for p in range(_NBUF):
        issue_gather(p, p)

    @pl.loop(0, _BPW // _NBUF)
    def _outer(b2):
        for p in range(_NBUF):
            b = b2 * _NBUF + p
            wait_gather(b, p)

            c0 = ctx_v[b, pl.ds(0, 16)]
            c1 = ctx_v[b, pl.ds(16, 16)]
            c2 = ctx_v[b, pl.ds(32, 16)]
            c3 = ctx_v[b, pl.ds(48, 16)]

            @pl.loop(0, _G, init_carry=jnp.full((16,), -1e30, jnp.float32))
            def _groups(g, m):
                v = jnp.zeros((16,), jnp.float32)
                for kk in range(16):
                    k = g * 16 + kk
                    acc = rows_v[p, k, pl.ds(0, 16)] * c0
                    acc = acc + rows_v[p, k, pl.ds(16, 16)] * c1
                    acc = acc + rows_v[p, k, pl.ds(32, 16)] * c2
                    acc = acc + rows_v[p, k, pl.ds(48, 16)] * c3
                    v = jnp.where(lane == kk, jnp.sum(acc), v)
                v = jnp.where(g * 16 + lane < _K, v, -1e30)
                scores_v[pl.ds(g * 16, 16)] = v
                return jnp.maximum(m, v)

            m = _groups
            mx = jnp.max(m)

            @pl.when(b + _NBUF < _BPW)
            def _():
                issue_gather(b + _NBUF, p)

            @pl.when(b2 > 0)
            def _():
                pltpu.make_async_copy(
                    out_vs[p].at[pl.ds(0, _K)],
                    out_hbm.at[base + b - _NBUF],
                    osems[p],
                ).wait()

            @pl.loop(0, _G, init_carry=jnp.zeros((16,), jnp.float32))
            def _expsum(g, tot):
                e = jnp.exp(scores_v[pl.ds(g * 16, 16)] - mx)
                out_vs[p][pl.ds(g * 16, 16)] = e
                return tot + e

            tvec = jnp.zeros((16,), jnp.float32) + jnp.sum(_expsum)

            @pl.loop(0, _G)
            def _scale(g):
                out_vs[p][pl.ds(g * 16, 16)] = out_vs[p][pl.ds(g * 16, 16)] / tvec

            pltpu.async_copy(
                out_vs[p].at[pl.ds(0, _K)],
                out_hbm.at[base + b],
                osems[p],
            )

    for p in range(_NBUF):
        pltpu.make_async_copy(
            out_vs[p].at[pl.ds(0, _K)],
            out_hbm.at[base + _BPW - _NBUF + p],
            osems[p],
        ).wait()


_sc_kernel = functools.partial(
    pl.kernel,
    out_type=jax.ShapeDtypeStruct((_B, _K), jnp.float32),
    mesh=plsc.VectorSubcoreMesh(core_axis_name="c", subcore_axis_name="s"),
    compiler_params=pltpu.CompilerParams(
        needs_layout_passes=False, use_tc_tiling_on_sc=False
    ),
    scratch_types=[
        pltpu.VMEM((_BPW, 2, _CH), jnp.int32),
        pltpu.VMEM((_BPW, _D), jnp.float32),
        pltpu.VMEM((_NBUF, _KP, _D), jnp.float32),
        pltpu.VMEM((_KP,), jnp.float32),
        pltpu.VMEM((_KP,), jnp.float32),
        pltpu.VMEM((_KP,), jnp.float32),
        pltpu.VMEM((_KP,), jnp.float32),
        pltpu.VMEM((_KP,), jnp.float32),
    ] + [pltpu.SemaphoreType.DMA] * 8,
)(_sc_body)


def kernel(x, A_k, W, b, table):
    ctx = _context_mlp(x, W, b)
    ak = A_k.astype(jnp.int32).reshape(_B, 2, _CH)
    return _sc_kernel(table, ak, ctx)


# speedup vs baseline: 1.6374x; 1.1002x over previous
import functools

import jax
import jax.numpy as jnp
from jax import lax
from jax.experimental import pallas as pl
from jax.experimental.pallas import tpu as pltpu
from jax.experimental.pallas import tpu_sc as plsc

_B = 4096
_DC = 128
_D = 64
_K = 200
_KP = 208
_G = _KP // 16
_CH = 100
_NC = 2
_NS = 16
_NW = _NC * _NS
_BPW = _B // _NW
_NBUF = 4


def _ctx_body(x_ref, w_ref, b_ref, o_ref):
    o_ref[...] = jnp.maximum(
        jnp.dot(x_ref[...], w_ref[...], preferred_element_type=jnp.float32)
        + b_ref[...],
        0.0,
    )


def _context_mlp(x, W, b):
    blk = 512
    return pl.pallas_call(
        _ctx_body,
        grid=(_B // blk,),
        in_specs=[
            pl.BlockSpec((blk, _DC), lambda i: (i, 0)),
            pl.BlockSpec((_DC, _D), lambda i: (0, 0)),
            pl.BlockSpec((1, _D), lambda i: (0, 0)),
        ],
        out_specs=pl.BlockSpec((blk, _D), lambda i: (i, 0)),
        out_shape=jax.ShapeDtypeStruct((_B, _D), jnp.float32),
    )(x, W, b.reshape(1, _D))


def _sc_body(table_hbm, ak_hbm, ctx_hbm, out_hbm,
             idx_v, ctx_v, rows_v, scores_v,
             out_v0, out_v1, out_v2, out_v3,
             gsem0, gsem1, gsem2, gsem3, osem0, osem1, osem2, osem3):
    out_vs = (out_v0, out_v1, out_v2, out_v3)
    wid = lax.axis_index("s") * _NC + lax.axis_index("c")
    base = wid * _BPW

    pltpu.sync_copy(ak_hbm.at[pl.ds(base, _BPW)], idx_v)
    pltpu.sync_copy(ctx_hbm.at[pl.ds(base, _BPW)], ctx_v)

    gsems = (gsem0, gsem1, gsem2, gsem3)
    osems = (osem0, osem1, osem2, osem3)
    lane = lax.iota(jnp.int32, 16)

    def issue_gather(b, p):
        for j in range(2):
            pltpu.async_copy(
                table_hbm.at[idx_v.at[b, j]],
                rows_v.at[p, pl.ds(j * _CH, _CH)],
                gsems[p],
            )

    def wait_gather(b, p):
        for j in range(2):
            pltpu.make_async_copy(
                table_hbm.at[idx_v.at[b, j]],
                rows_v.at[p, pl.ds(j * _CH, _CH)],
                gsems[p],
            ).wait()

    for p in range(_NBUF):
        issue_gather(p, p)

    @pl.loop(0, _BPW // _NBUF)
    def _outer(b2):
        for p in range(_NBUF):
            b = b2 * _NBUF + p
            wait_gather(b, p)

            c0 = ctx_v[b, pl.ds(0, 16)]
            c1 = ctx_v[b, pl.ds(16, 16)]
            c2 = ctx_v[b, pl.ds(32, 16)]
            c3 = ctx_v[b, pl.ds(48, 16)]

            @pl.loop(0, _G, init_carry=jnp.full((16,), -1e30, jnp.float32))
            def _groups(g, m):
                v = jnp.zeros((16,), jnp.float32)
                for kk in range(16):
                    k = g * 16 + kk
                    acc = rows_v[p, k, pl.ds(0, 16)] * c0
                    acc = acc + rows_v[p, k, pl.ds(16, 16)] * c1
                    acc = acc + rows_v[p, k, pl.ds(32, 16)] * c2
                    acc = acc + rows_v[p, k, pl.ds(48, 16)] * c3
                    v = jnp.where(lane == kk, jnp.sum(acc), v)
                v = jnp.where(g * 16 + lane < _K, v, -1e30)
                scores_v[pl.ds(g * 16, 16)] = v
                return jnp.maximum(m, v)

            m = _groups
            mx = jnp.max(m)

            @pl.when(b + _NBUF < _BPW)
            def _():
                issue_gather(b + _NBUF, p)

            @pl.when(b2 > 0)
            def _():
                pltpu.make_async_copy(
                    out_vs[p].at[pl.ds(0, _K)],
                    out_hbm.at[base + b - _NBUF],
                    osems[p],
                ).wait()

            @pl.loop(0, _G, init_carry=jnp.zeros((16,), jnp.float32))
            def _expsum(g, tot):
                e = jnp.exp(scores_v[pl.ds(g * 16, 16)] - mx)
                out_vs[p][pl.ds(g * 16, 16)] = e
                return tot + e

            tvec = jnp.zeros((16,), jnp.float32) + jnp.sum(_expsum)

            @pl.loop(0, _G)
            def _scale(g):
                out_vs[p][pl.ds(g * 16, 16)] = out_vs[p][pl.ds(g * 16, 16)] / tvec

            pltpu.async_copy(
                out_vs[p].at[pl.ds(0, _K)],
                out_hbm.at[base + b],
                osems[p],
            )

    for p in range(_NBUF):
        pltpu.make_async_copy(
            out_vs[p].at[pl.ds(0, _K)],
            out_hbm.at[base + _BPW - _NBUF + p],
            osems[p],
        ).wait()


_sc_kernel = functools.partial(
    pl.kernel,
    out_type=jax.ShapeDtypeStruct((_B, _K), jnp.float32),
    mesh=plsc.VectorSubcoreMesh(core_axis_name="c", subcore_axis_name="s"),
    compiler_params=pltpu.CompilerParams(
        needs_layout_passes=False, use_tc_tiling_on_sc=False
    ),
    scratch_types=[
        pltpu.VMEM((_BPW, 2, _CH), jnp.int32),
        pltpu.VMEM((_BPW, _D), jnp.float32),
        pltpu.VMEM((_NBUF, _KP, _D), jnp.float32),
        pltpu.VMEM((_KP,), jnp.float32),
        pltpu.VMEM((_KP,), jnp.float32),
        pltpu.VMEM((_KP,), jnp.float32),
        pltpu.VMEM((_KP,), jnp.float32),
        pltpu.VMEM((_KP,), jnp.float32),
    ] + [pltpu.SemaphoreType.DMA] * 8,
)(_sc_body)


def kernel(x, A_k, W, b, table):
    ctx = _context_mlp(x, W, b)
    ak = (A_k.astype(jnp.int32) * 2).reshape(_B, 2, _CH)
    tp = jnp.pad(table, ((0, 0), (0, _D))).reshape(2 * 1000000, _D)
    return _sc_kernel(tp, ak, ctx)
